# baseline (speedup 1.0000x reference)
import functools

import jax
import jax.numpy as jnp
from jax import lax
from jax.experimental import pallas as pl
from jax.experimental.pallas import tpu as pltpu
from jax.experimental.pallas import tpu_sc as plsc

N_NODES = 10000
N_PAD = 10112
E_EDGES = 320000
CHUNK = 128
CH_TOT = E_EDGES // CHUNK
BASE_CH = CH_TOT // 32
EXTRA_W = CH_TOT - 32 * BASE_CH
RING_CH = 76
D_IN = 128
D_HID = 32
D_CLS = 10
D_CLS_PAD = 16
ZROWS = N_PAD // 16

_mesh = plsc.VectorSubcoreMesh(core_axis_name="c", subcore_axis_name="s")
_sc_params = pltpu.CompilerParams(use_tc_tiling_on_sc=False)


def _load_idx(idx_hbm, idx_v, wid):
    pltpu.sync_copy(idx_hbm.at[pl.ds(wid * BASE_CH, BASE_CH)],
                    idx_v.at[pl.ds(0, BASE_CH)])

    @pl.when(wid < EXTRA_W)
    def _():
        pltpu.sync_copy(idx_hbm.at[pl.ds(32 * BASE_CH + wid, 1)],
                        idx_v.at[pl.ds(BASE_CH, 1)])


@functools.partial(
    pl.kernel,
    mesh=_mesh,
    compiler_params=_sc_params,
    out_type=jax.ShapeDtypeStruct((2, N_PAD, 8), jnp.float32),
    scratch_types=[
        pltpu.VMEM((BASE_CH + 1, CHUNK), jnp.int32),
        pltpu.VMEM((CHUNK, 16), jnp.float32),
        pltpu.VMEM((ZROWS, 16), jnp.float32),
        pltpu.VMEM_SHARED((N_PAD, 16), jnp.float32),
        pltpu.SemaphoreType.DMA,
    ],
)
def _sc_degree(dst_hbm, out_hbm, idx_v, ones_v, zero_v, acc_sh, sem):
    c = lax.axis_index("c")
    s = lax.axis_index("s")
    wid = c * 16 + s
    nch = BASE_CH + jnp.where(wid < EXTRA_W, 1, 0)

    @pl.loop(0, CHUNK)
    def _(r):
        ones_v[r, :] = jnp.full((16,), 1.0, jnp.float32)

    @pl.loop(0, ZROWS)
    def _(r):
        zero_v[r, :] = jnp.zeros((16,), jnp.float32)

    pltpu.sync_copy(zero_v, acc_sh.at[pl.ds(s * ZROWS, ZROWS)])
    _load_idx(dst_hbm, idx_v, wid)
    plsc.subcore_barrier()

    DEPTH = 8
    for j in range(DEPTH):
        pltpu.async_copy(ones_v, acc_sh.at[idx_v.at[j]], sem, add=True)

    @pl.loop(DEPTH, RING_CH)
    def _(j):
        pltpu.make_async_copy(ones_v, acc_sh.at[idx_v.at[j]], sem).wait()
        pltpu.async_copy(ones_v, acc_sh.at[idx_v.at[j]], sem, add=True)

    for j in range(DEPTH):
        pltpu.make_async_copy(ones_v, acc_sh.at[idx_v.at[j]], sem).wait()

    for k in range(RING_CH, BASE_CH + 1):
        @pl.when(k < nch)
        def _():
            pltpu.sync_copy(ones_v, acc_sh.at[idx_v.at[k]], add=True)

    plsc.subcore_barrier()
    pltpu.sync_copy(acc_sh.at[pl.ds(s * ZROWS, ZROWS), pl.ds(0, 8)],
                    out_hbm.at[c, pl.ds(s * ZROWS, ZROWS)])


def _make_sc_agg(depth):
    @functools.partial(
        pl.kernel,
        mesh=_mesh,
        compiler_params=_sc_params,
        out_type=jax.ShapeDtypeStruct((2, N_PAD, depth), jnp.float32),
        scratch_types=[
            pltpu.VMEM((BASE_CH + 1, CHUNK), jnp.int32),
            pltpu.VMEM((BASE_CH + 1, CHUNK), jnp.int32),
            pltpu.VMEM((4, CHUNK, depth), jnp.float32),
            pltpu.VMEM((ZROWS, depth), jnp.float32),
            pltpu.VMEM_SHARED((N_PAD, depth), jnp.float32),
            pltpu.SemaphoreType.DMA,
            pltpu.SemaphoreType.DMA,
            pltpu.SemaphoreType.DMA,
            pltpu.SemaphoreType.DMA,
            pltpu.SemaphoreType.DMA,
            pltpu.SemaphoreType.DMA,
            pltpu.SemaphoreType.DMA,
            pltpu.SemaphoreType.DMA,
        ],
    )
    def _agg(g_hbm, src_hbm, dst_hbm, out_hbm,
             src_v, dst_v, bufs, zero_v, acc_sh,
             g0, g1, g2, g3, s0, s1, s2, s3):
        c = lax.axis_index("c")
        s = lax.axis_index("s")
        wid = c * 16 + s
        nch = BASE_CH + jnp.where(wid < EXTRA_W, 1, 0)
        gsem = (g0, g1, g2, g3)
        ssem = (s0, s1, s2, s3)

        @pl.loop(0, ZROWS)
        def _(r):
            for t in range(depth // 16):
                zero_v[r, pl.ds(t * 16, 16)] = jnp.zeros((16,), jnp.float32)

        pltpu.sync_copy(zero_v, acc_sh.at[pl.ds(s * ZROWS, ZROWS)])
        _load_idx(src_hbm, src_v, wid)
        _load_idx(dst_hbm, dst_v, wid)

        for b in range(4):
            pltpu.async_copy(g_hbm.at[src_v.at[b]], bufs.at[b], gsem[b])
        plsc.subcore_barrier()

        @pl.loop(0, RING_CH, step=4)
        def _(j):
            for b in range(4):
                pltpu.make_async_copy(
                    g_hbm.at[src_v.at[j + b]], bufs.at[b], gsem[b]).wait()
                pltpu.async_copy(
                    bufs.at[b], acc_sh.at[dst_v.at[j + b]], ssem[b], add=True)
            for b in range(4):
                pltpu.make_async_copy(
                    bufs.at[b], acc_sh.at[dst_v.at[j + b]], ssem[b]).wait()

                @pl.when(j + 4 + b < RING_CH)
                def _():
                    pltpu.async_copy(
                        g_hbm.at[src_v.at[j + 4 + b]], bufs.at[b], gsem[b])

        for k in range(RING_CH, BASE_CH + 1):
            @pl.when(k < nch)
            def _():
                pltpu.sync_copy(g_hbm.at[src_v.at[k]], bufs.at[0])
                pltpu.sync_copy(bufs.at[0], acc_sh.at[dst_v.at[k]], add=True)

        plsc.subcore_barrier()
        pltpu.sync_copy(acc_sh.at[pl.ds(s * ZROWS, ZROWS)],
                        out_hbm.at[c, pl.ds(s * ZROWS, ZROWS)])

    return _agg


_sc_agg32 = _make_sc_agg(D_HID)
_sc_agg16 = _make_sc_agg(D_CLS_PAD)


def _tc_mm1(x, w1):
    def body(x_ref, w_ref, o_ref):
        o_ref[pl.ds(0, N_NODES), :] = jnp.dot(
            x_ref[...], w_ref[...], preferred_element_type=jnp.float32)
        o_ref[pl.ds(N_NODES, N_PAD - N_NODES), :] = jnp.zeros(
            (N_PAD - N_NODES, D_HID), jnp.float32)

    return pl.pallas_call(
        body,
        out_shape=jax.ShapeDtypeStruct((N_PAD, D_HID), jnp.float32),
    )(x, w1)


_GB = 8
_BR = N_PAD // _GB


def _tc_scale(degp, h1):
    def body(d_ref, h_ref, dis_ref, g_ref):
        d = d_ref[...]
        deg = d[0, :, 0:1] + d[1, :, 0:1] + 1.0
        dis = lax.rsqrt(deg)
        dis_ref[...] = dis
        g_ref[...] = h_ref[...] * dis

    return pl.pallas_call(
        body,
        grid=(_GB,),
        in_specs=[
            pl.BlockSpec((2, _BR, 8), lambda i: (0, i, 0)),
            pl.BlockSpec((_BR, D_HID), lambda i: (i, 0)),
        ],
        out_specs=[
            pl.BlockSpec((_BR, 1), lambda i: (i, 0)),
            pl.BlockSpec((_BR, D_HID), lambda i: (i, 0)),
        ],
        out_shape=[
            jax.ShapeDtypeStruct((N_PAD, 1), jnp.float32),
            jax.ShapeDtypeStruct((N_PAD, D_HID), jnp.float32),
        ],
    )(degp, h1)


def _tc_layer2(aggp, g1, dis, b1r, w2p):
    def body(a_ref, g_ref, dis_ref, b_ref, w_ref, o_ref):
        a = a_ref[...]
        dis = dis_ref[...]
        u = (a[0] + a[1] + g_ref[...]) * dis + b_ref[...]
        t = jnp.tanh(u)
        h2 = jnp.dot(t, w_ref[...], preferred_element_type=jnp.float32)
        o_ref[...] = h2 * dis

    return pl.pallas_call(
        body,
        grid=(_GB,),
        in_specs=[
            pl.BlockSpec((2, _BR, D_HID), lambda i: (0, i, 0)),
            pl.BlockSpec((_BR, D_HID), lambda i: (i, 0)),
            pl.BlockSpec((_BR, 1), lambda i: (i, 0)),
            pl.BlockSpec((1, D_HID), lambda i: (0, 0)),
            pl.BlockSpec((D_HID, D_CLS_PAD), lambda i: (0, 0)),
        ],
        out_specs=pl.BlockSpec((_BR, D_CLS_PAD), lambda i: (i, 0)),
        out_shape=jax.ShapeDtypeStruct((N_PAD, D_CLS_PAD), jnp.float32),
    )(aggp, g1, dis, b1r, w2p)


def _tc_out(aggp, g2, dis, b2r):
    def body(a_ref, g_ref, dis_ref, b_ref, o_ref):
        a = a_ref[...]
        u = (a[0] + a[1] + g_ref[...]) * dis_ref[...] + b_ref[...]
        logits = u[:, :D_CLS]
        m = jnp.max(logits, axis=1, keepdims=True)
        sh = logits - m
        lse = jnp.log(jnp.sum(jnp.exp(sh), axis=1, keepdims=True))
        o_ref[...] = sh - lse

    return pl.pallas_call(
        body,
        grid=(10,),
        in_specs=[
            pl.BlockSpec((2, 1000, D_CLS_PAD), lambda i: (0, i, 0)),
            pl.BlockSpec((1000, D_CLS_PAD), lambda i: (i, 0)),
            pl.BlockSpec((1000, 1), lambda i: (i, 0)),
            pl.BlockSpec((1, D_CLS_PAD), lambda i: (0, 0)),
        ],
        out_specs=pl.BlockSpec((1000, D_CLS), lambda i: (i, 0)),
        out_shape=jax.ShapeDtypeStruct((N_NODES, D_CLS), jnp.float32),
    )(aggp, g2, dis, b2r)


def kernel(x, edge_index, W1, b1, W2, b2):
    src2 = edge_index[0].reshape(CH_TOT, CHUNK)
    dst2 = edge_index[1].reshape(CH_TOT, CHUNK)
    w2p = jnp.pad(W2, ((0, 0), (0, D_CLS_PAD - D_CLS)))
    b1r = b1.reshape(1, D_HID)
    b2r = jnp.pad(b2, (0, D_CLS_PAD - D_CLS)).reshape(1, D_CLS_PAD)

    h1 = _tc_mm1(x, W1)
    degp = _sc_degree(dst2)
    dis, g1 = _tc_scale(degp, h1)
    agg1 = _sc_agg32(g1, src2, dst2)
    g2 = _tc_layer2(agg1, g1, dis, b1r, w2p)
    agg2 = _sc_agg16(g2, src2, dst2)
    return _tc_out(agg2, g2, dis, b2r)


# speedup vs baseline: 1.0437x; 1.0437x over previous
import functools

import jax
import jax.numpy as jnp
from jax import lax
from jax.experimental import pallas as pl
from jax.experimental.pallas import tpu as pltpu
from jax.experimental.pallas import tpu_sc as plsc

N_NODES = 10000
N_PAD = 10112
E_EDGES = 320000
CHUNK = 128
CH_TOT = E_EDGES // CHUNK
BASE_CH = CH_TOT // 32
EXTRA_W = CH_TOT - 32 * BASE_CH
RING_CH = 76
D_IN = 128
D_HID = 32
D_CLS = 10
D_CLS_PAD = 16
ZROWS = N_PAD // 16

_mesh = plsc.VectorSubcoreMesh(core_axis_name="c", subcore_axis_name="s")
_sc_params = pltpu.CompilerParams(use_tc_tiling_on_sc=False)


def _load_idx(idx_hbm, idx_v, wid):
    pltpu.sync_copy(idx_hbm.at[pl.ds(wid * BASE_CH, BASE_CH)],
                    idx_v.at[pl.ds(0, BASE_CH)])

    @pl.when(wid < EXTRA_W)
    def _():
        pltpu.sync_copy(idx_hbm.at[pl.ds(32 * BASE_CH + wid, 1)],
                        idx_v.at[pl.ds(BASE_CH, 1)])


@functools.partial(
    pl.kernel,
    mesh=_mesh,
    compiler_params=_sc_params,
    out_type=jax.ShapeDtypeStruct((2, N_PAD, 16), jnp.float32),
    scratch_types=[
        pltpu.VMEM((BASE_CH + 1, CHUNK), jnp.int32),
        pltpu.VMEM((CHUNK, 16), jnp.float32),
        pltpu.VMEM((ZROWS, 16), jnp.float32),
        pltpu.VMEM_SHARED((N_PAD, 16), jnp.float32),
        pltpu.SemaphoreType.DMA,
    ],
)
def _sc_degree(dst_hbm, out_hbm, idx_v, ones_v, zero_v, acc_sh, sem):
    c = lax.axis_index("c")
    s = lax.axis_index("s")
    wid = c * 16 + s
    nch = BASE_CH + jnp.where(wid < EXTRA_W, 1, 0)

    @pl.loop(0, CHUNK)
    def _(r):
        ones_v[r, :] = jnp.full((16,), 1.0, jnp.float32)

    @pl.loop(0, ZROWS)
    def _(r):
        zero_v[r, :] = jnp.zeros((16,), jnp.float32)

    pltpu.sync_copy(zero_v, acc_sh.at[pl.ds(s * ZROWS, ZROWS)])
    _load_idx(dst_hbm, idx_v, wid)
    plsc.subcore_barrier()

    DEPTH = 8
    for j in range(DEPTH):
        pltpu.async_copy(ones_v, acc_sh.at[idx_v.at[j]], sem, add=True)

    @pl.loop(DEPTH, RING_CH)
    def _(j):
        pltpu.make_async_copy(ones_v, acc_sh.at[idx_v.at[j]], sem).wait()
        pltpu.async_copy(ones_v, acc_sh.at[idx_v.at[j]], sem, add=True)

    for j in range(DEPTH):
        pltpu.make_async_copy(ones_v, acc_sh.at[idx_v.at[j]], sem).wait()

    for k in range(RING_CH, BASE_CH + 1):
        @pl.when(k < nch)
        def _():
            pltpu.sync_copy(ones_v, acc_sh.at[idx_v.at[k]], add=True)

    plsc.subcore_barrier()
    pltpu.sync_copy(acc_sh.at[pl.ds(s * ZROWS, ZROWS)],
                    out_hbm.at[c, pl.ds(s * ZROWS, ZROWS)])


def _make_sc_agg(depth):
    @functools.partial(
        pl.kernel,
        mesh=_mesh,
        compiler_params=_sc_params,
        out_type=jax.ShapeDtypeStruct((2, N_PAD, depth), jnp.float32),
        scratch_types=[
            pltpu.VMEM((BASE_CH + 1, CHUNK), jnp.int32),
            pltpu.VMEM((BASE_CH + 1, CHUNK), jnp.int32),
            pltpu.VMEM((4, CHUNK, depth), jnp.float32),
            pltpu.VMEM((ZROWS, depth), jnp.float32),
            pltpu.VMEM_SHARED((N_PAD, depth), jnp.float32),
            pltpu.SemaphoreType.DMA,
            pltpu.SemaphoreType.DMA,
            pltpu.SemaphoreType.DMA,
            pltpu.SemaphoreType.DMA,
            pltpu.SemaphoreType.DMA,
            pltpu.SemaphoreType.DMA,
            pltpu.SemaphoreType.DMA,
            pltpu.SemaphoreType.DMA,
        ],
    )
    def _agg(g_hbm, src_hbm, dst_hbm, out_hbm,
             src_v, dst_v, bufs, zero_v, acc_sh,
             g0, g1, g2, g3, s0, s1, s2, s3):
        c = lax.axis_index("c")
        s = lax.axis_index("s")
        wid = c * 16 + s
        nch = BASE_CH + jnp.where(wid < EXTRA_W, 1, 0)
        gsem = (g0, g1, g2, g3)
        ssem = (s0, s1, s2, s3)

        @pl.loop(0, ZROWS)
        def _(r):
            for t in range(depth // 16):
                zero_v[r, pl.ds(t * 16, 16)] = jnp.zeros((16,), jnp.float32)

        pltpu.sync_copy(zero_v, acc_sh.at[pl.ds(s * ZROWS, ZROWS)])
        _load_idx(src_hbm, src_v, wid)
        _load_idx(dst_hbm, dst_v, wid)

        for b in range(4):
            pltpu.async_copy(g_hbm.at[src_v.at[b]], bufs.at[b], gsem[b])
        plsc.subcore_barrier()

        @pl.loop(0, RING_CH, step=4)
        def _(j):
            for b in range(4):
                pltpu.make_async_copy(
                    g_hbm.at[src_v.at[j + b]], bufs.at[b], gsem[b]).wait()
                pltpu.async_copy(
                    bufs.at[b], acc_sh.at[dst_v.at[j + b]], ssem[b], add=True)
            for b in range(4):
                pltpu.make_async_copy(
                    bufs.at[b], acc_sh.at[dst_v.at[j + b]], ssem[b]).wait()

                @pl.when(j + 4 + b < RING_CH)
                def _():
                    pltpu.async_copy(
                        g_hbm.at[src_v.at[j + 4 + b]], bufs.at[b], gsem[b])

        for k in range(RING_CH, BASE_CH + 1):
            @pl.when(k < nch)
            def _():
                pltpu.sync_copy(g_hbm.at[src_v.at[k]], bufs.at[0])
                pltpu.sync_copy(bufs.at[0], acc_sh.at[dst_v.at[k]], add=True)

        plsc.subcore_barrier()
        pltpu.sync_copy(acc_sh.at[pl.ds(s * ZROWS, ZROWS)],
                        out_hbm.at[c, pl.ds(s * ZROWS, ZROWS)])

    return _agg


_sc_agg32 = _make_sc_agg(D_HID)
_sc_agg16 = _make_sc_agg(D_CLS_PAD)


def _tc_mm1(x, w1):
    def body(x_ref, w_ref, o_ref):
        o_ref[pl.ds(0, N_NODES), :] = jnp.dot(
            x_ref[...], w_ref[...], preferred_element_type=jnp.float32)
        o_ref[pl.ds(N_NODES, N_PAD - N_NODES), :] = jnp.zeros(
            (N_PAD - N_NODES, D_HID), jnp.float32)

    return pl.pallas_call(
        body,
        out_shape=jax.ShapeDtypeStruct((N_PAD, D_HID), jnp.float32),
    )(x, w1)


_GB = 8
_BR = N_PAD // _GB


def _tc_scale(degp, h1):
    def body(d_ref, h_ref, dis_ref, g_ref):
        d = d_ref[...]
        deg = d[0, :, 0:1] + d[1, :, 0:1] + 1.0
        dis = lax.rsqrt(deg)
        dis_ref[...] = dis
        g_ref[...] = h_ref[...] * dis

    return pl.pallas_call(
        body,
        out_shape=[
            jax.ShapeDtypeStruct((N_PAD, 1), jnp.float32),
            jax.ShapeDtypeStruct((N_PAD, D_HID), jnp.float32),
        ],
    )(degp, h1)


def _tc_layer2(aggp, g1, dis, b1r, w2p):
    def body(a_ref, g_ref, dis_ref, b_ref, w_ref, o_ref):
        a = a_ref[...]
        dis = dis_ref[...]
        u = (a[0] + a[1] + g_ref[...]) * dis + b_ref[...]
        t = jnp.tanh(u)
        h2 = jnp.dot(t, w_ref[...], preferred_element_type=jnp.float32)
        o_ref[...] = h2 * dis

    return pl.pallas_call(
        body,
        out_shape=jax.ShapeDtypeStruct((N_PAD, D_CLS_PAD), jnp.float32),
    )(aggp, g1, dis, b1r, w2p)


def _tc_out(aggp, g2, dis, b2r):
    def body(a_ref, g_ref, dis_ref, b_ref, o_ref):
        a = a_ref[...]
        u = (a[0] + a[1] + g_ref[...]) * dis_ref[...] + b_ref[...]
        logits = u[:N_NODES, :D_CLS]
        m = jnp.max(logits, axis=1, keepdims=True)
        sh = logits - m
        lse = jnp.log(jnp.sum(jnp.exp(sh), axis=1, keepdims=True))
        o_ref[...] = sh - lse

    return pl.pallas_call(
        body,
        out_shape=jax.ShapeDtypeStruct((N_NODES, D_CLS), jnp.float32),
    )(aggp, g2, dis, b2r)


def kernel(x, edge_index, W1, b1, W2, b2):
    src2 = edge_index[0].reshape(CH_TOT, CHUNK)
    dst2 = edge_index[1].reshape(CH_TOT, CHUNK)
    w2p = jnp.pad(W2, ((0, 0), (0, D_CLS_PAD - D_CLS)))
    b1r = b1.reshape(1, D_HID)
    b2r = jnp.pad(b2, (0, D_CLS_PAD - D_CLS)).reshape(1, D_CLS_PAD)

    h1 = _tc_mm1(x, W1)
    degp = _sc_degree(dst2)
    dis, g1 = _tc_scale(degp, h1)
    agg1 = _sc_agg32(g1, src2, dst2)
    g2 = _tc_layer2(agg1, g1, dis, b1r, w2p)
    agg2 = _sc_agg16(g2, src2, dst2)
    return _tc_out(agg2, g2, dis, b2r)


# speedup vs baseline: 1.1113x; 1.0648x over previous
import functools

import jax
import jax.numpy as jnp
from jax import lax
from jax.experimental import pallas as pl
from jax.experimental.pallas import tpu as pltpu
from jax.experimental.pallas import tpu_sc as plsc

N_NODES = 10000
N_PAD = 10112
E_EDGES = 320000
CHUNK = 128
CH_TOT = E_EDGES // CHUNK
BASE_CH = CH_TOT // 32
EXTRA_W = CH_TOT - 32 * BASE_CH
NBUF = 8
RING_CH = 72
D_IN = 128
D_HID = 32
D_CLS = 10
D_CLS_PAD = 16
ZROWS = N_PAD // 16

_mesh = plsc.VectorSubcoreMesh(core_axis_name="c", subcore_axis_name="s")
_sc_params = pltpu.CompilerParams(use_tc_tiling_on_sc=False)


def _load_idx(idx_hbm, idx_v, wid):
    pltpu.sync_copy(idx_hbm.at[pl.ds(wid * BASE_CH, BASE_CH)],
                    idx_v.at[pl.ds(0, BASE_CH)])

    @pl.when(wid < EXTRA_W)
    def _():
        pltpu.sync_copy(idx_hbm.at[pl.ds(32 * BASE_CH + wid, 1)],
                        idx_v.at[pl.ds(BASE_CH, 1)])


@functools.partial(
    pl.kernel,
    mesh=_mesh,
    compiler_params=_sc_params,
    out_type=jax.ShapeDtypeStruct((2, N_PAD, 16), jnp.float32),
    scratch_types=[
        pltpu.VMEM((BASE_CH + 1, CHUNK), jnp.int32),
        pltpu.VMEM((CHUNK, 16), jnp.float32),
        pltpu.VMEM((ZROWS, 16), jnp.float32),
        pltpu.VMEM_SHARED((N_PAD, 16), jnp.float32),
        pltpu.SemaphoreType.DMA,
    ],
)
def _sc_degree(dst_hbm, out_hbm, idx_v, ones_v, zero_v, acc_sh, sem):
    c = lax.axis_index("c")
    s = lax.axis_index("s")
    wid = c * 16 + s
    nch = BASE_CH + jnp.where(wid < EXTRA_W, 1, 0)

    @pl.loop(0, CHUNK)
    def _(r):
        ones_v[r, :] = jnp.full((16,), 1.0, jnp.float32)

    @pl.loop(0, ZROWS)
    def _(r):
        zero_v[r, :] = jnp.zeros((16,), jnp.float32)

    pltpu.sync_copy(zero_v, acc_sh.at[pl.ds(s * ZROWS, ZROWS)])
    _load_idx(dst_hbm, idx_v, wid)
    plsc.subcore_barrier()

    DEPTH = 8
    for j in range(DEPTH):
        pltpu.async_copy(ones_v, acc_sh.at[idx_v.at[j]], sem, add=True)

    @pl.loop(DEPTH, RING_CH)
    def _(j):
        pltpu.make_async_copy(ones_v, acc_sh.at[idx_v.at[j]], sem).wait()
        pltpu.async_copy(ones_v, acc_sh.at[idx_v.at[j]], sem, add=True)

    for j in range(DEPTH):
        pltpu.make_async_copy(ones_v, acc_sh.at[idx_v.at[j]], sem).wait()

    for k in range(RING_CH, BASE_CH + 1):
        @pl.when(k < nch)
        def _():
            pltpu.sync_copy(ones_v, acc_sh.at[idx_v.at[k]], add=True)

    plsc.subcore_barrier()
    pltpu.sync_copy(acc_sh.at[pl.ds(s * ZROWS, ZROWS)],
                    out_hbm.at[c, pl.ds(s * ZROWS, ZROWS)])


def _make_sc_agg(depth):
    @functools.partial(
        pl.kernel,
        mesh=_mesh,
        compiler_params=_sc_params,
        out_type=jax.ShapeDtypeStruct((2, N_PAD, depth), jnp.float32),
        scratch_types=[
            pltpu.VMEM((BASE_CH + 1, CHUNK), jnp.int32),
            pltpu.VMEM((BASE_CH + 1, CHUNK), jnp.int32),
            pltpu.VMEM((NBUF, CHUNK, depth), jnp.float32),
            pltpu.VMEM((ZROWS, depth), jnp.float32),
            pltpu.VMEM_SHARED((N_PAD, depth), jnp.float32),
        ] + [pltpu.SemaphoreType.DMA] * (2 * NBUF),
    )
    def _agg(g_hbm, src_hbm, dst_hbm, out_hbm,
             src_v, dst_v, bufs, zero_v, acc_sh, *sems):
        c = lax.axis_index("c")
        s = lax.axis_index("s")
        wid = c * 16 + s
        nch = BASE_CH + jnp.where(wid < EXTRA_W, 1, 0)
        gsem = sems[:NBUF]
        ssem = sems[NBUF:]

        @pl.loop(0, ZROWS)
        def _(r):
            for t in range(depth // 16):
                zero_v[r, pl.ds(t * 16, 16)] = jnp.zeros((16,), jnp.float32)

        pltpu.sync_copy(zero_v, acc_sh.at[pl.ds(s * ZROWS, ZROWS)])
        _load_idx(src_hbm, src_v, wid)
        _load_idx(dst_hbm, dst_v, wid)

        for b in range(NBUF):
            pltpu.async_copy(g_hbm.at[src_v.at[b]], bufs.at[b], gsem[b])
        plsc.subcore_barrier()

        @pl.loop(0, RING_CH, step=NBUF)
        def _(j):
            for b in range(NBUF):
                pltpu.make_async_copy(
                    g_hbm.at[src_v.at[j + b]], bufs.at[b], gsem[b]).wait()
                pltpu.async_copy(
                    bufs.at[b], acc_sh.at[dst_v.at[j + b]], ssem[b], add=True)
            for b in range(NBUF):
                pltpu.make_async_copy(
                    bufs.at[b], acc_sh.at[dst_v.at[j + b]], ssem[b]).wait()

                @pl.when(j + NBUF + b < RING_CH)
                def _():
                    pltpu.async_copy(
                        g_hbm.at[src_v.at[j + NBUF + b]], bufs.at[b], gsem[b])

        for k in range(RING_CH, BASE_CH + 1):
            @pl.when(k < nch)
            def _():
                pltpu.async_copy(g_hbm.at[src_v.at[k]],
                                 bufs.at[k - RING_CH], gsem[k - RING_CH])
        for k in range(RING_CH, BASE_CH + 1):
            @pl.when(k < nch)
            def _():
                pltpu.make_async_copy(g_hbm.at[src_v.at[k]],
                                      bufs.at[k - RING_CH],
                                      gsem[k - RING_CH]).wait()
                pltpu.sync_copy(bufs.at[k - RING_CH],
                                acc_sh.at[dst_v.at[k]], add=True)

        plsc.subcore_barrier()
        pltpu.sync_copy(acc_sh.at[pl.ds(s * ZROWS, ZROWS)],
                        out_hbm.at[c, pl.ds(s * ZROWS, ZROWS)])

    return _agg


_sc_agg32 = _make_sc_agg(D_HID)
_sc_agg16 = _make_sc_agg(D_CLS_PAD)


def _tc_mm1(x, w1):
    def body(x_ref, w_ref, o_ref):
        o_ref[pl.ds(0, N_NODES), :] = jnp.dot(
            x_ref[...], w_ref[...], preferred_element_type=jnp.float32)
        o_ref[pl.ds(N_NODES, N_PAD - N_NODES), :] = jnp.zeros(
            (N_PAD - N_NODES, D_HID), jnp.float32)

    return pl.pallas_call(
        body,
        out_shape=jax.ShapeDtypeStruct((N_PAD, D_HID), jnp.float32),
    )(x, w1)


_GB = 8
_BR = N_PAD // _GB


def _tc_scale(degp, h1):
    def body(d_ref, h_ref, dis_ref, g_ref):
        d = d_ref[...]
        deg = d[0, :, 0:1] + d[1, :, 0:1] + 1.0
        dis = lax.rsqrt(deg)
        dis_ref[...] = dis
        g_ref[...] = h_ref[...] * dis

    return pl.pallas_call(
        body,
        out_shape=[
            jax.ShapeDtypeStruct((N_PAD, 1), jnp.float32),
            jax.ShapeDtypeStruct((N_PAD, D_HID), jnp.float32),
        ],
    )(degp, h1)


def _tc_layer2(aggp, g1, dis, b1r, w2p):
    def body(a_ref, g_ref, dis_ref, b_ref, w_ref, o_ref):
        a = a_ref[...]
        dis = dis_ref[...]
        u = (a[0] + a[1] + g_ref[...]) * dis + b_ref[...]
        t = jnp.tanh(u)
        h2 = jnp.dot(t, w_ref[...], preferred_element_type=jnp.float32)
        o_ref[...] = h2 * dis

    return pl.pallas_call(
        body,
        out_shape=jax.ShapeDtypeStruct((N_PAD, D_CLS_PAD), jnp.float32),
    )(aggp, g1, dis, b1r, w2p)


def _tc_out(aggp, g2, dis, b2r):
    def body(a_ref, g_ref, dis_ref, b_ref, o_ref):
        a = a_ref[...]
        u = (a[0] + a[1] + g_ref[...]) * dis_ref[...] + b_ref[...]
        logits = u[:N_NODES, :D_CLS]
        m = jnp.max(logits, axis=1, keepdims=True)
        sh = logits - m
        lse = jnp.log(jnp.sum(jnp.exp(sh), axis=1, keepdims=True))
        o_ref[...] = sh - lse

    return pl.pallas_call(
        body,
        out_shape=jax.ShapeDtypeStruct((N_NODES, D_CLS), jnp.float32),
    )(aggp, g2, dis, b2r)


def kernel(x, edge_index, W1, b1, W2, b2):
    src2 = edge_index[0].reshape(CH_TOT, CHUNK)
    dst2 = edge_index[1].reshape(CH_TOT, CHUNK)
    w2p = jnp.pad(W2, ((0, 0), (0, D_CLS_PAD - D_CLS)))
    b1r = b1.reshape(1, D_HID)
    b2r = jnp.pad(b2, (0, D_CLS_PAD - D_CLS)).reshape(1, D_CLS_PAD)

    h1 = _tc_mm1(x, W1)
    degp = _sc_degree(dst2)
    dis, g1 = _tc_scale(degp, h1)
    agg1 = _sc_agg32(g1, src2, dst2)
    g2 = _tc_layer2(agg1, g1, dis, b1r, w2p)
    agg2 = _sc_agg16(g2, src2, dst2)
    return _tc_out(agg2, g2, dis, b2r)
